# baseline (speedup 1.0000x reference)
import functools

import jax
import jax.numpy as jnp
from jax import lax
from jax.experimental import pallas as pl
from jax.experimental.pallas import tpu as pltpu
from jax.experimental.pallas import tpu_sc as plsc

B = 4
L = 4096
D = 1024
TABLE = 4096

NC = 2
NS = 16
NW = NC * NS
TOTAL = B * L
B_PER_W = TOTAL // NW
W_PER_ROW = L // B_PER_W
FILL = 4
NFILL = B_PER_W // FILL
NGRP = B_PER_W // 16


@functools.cache
def _build_fused():
    return pl.kernel(
        _fused_body,
        out_type=jax.ShapeDtypeStruct((TOTAL, D), jnp.float32),
        mesh=plsc.VectorSubcoreMesh(core_axis_name="c", subcore_axis_name="s"),
        scratch_types=[
            pltpu.VMEM((1, B_PER_W), jnp.int32),
            pltpu.VMEM((16,), jnp.int32),
            pltpu.VMEM((16,), jnp.int32),
            pltpu.VMEM((NS, 16), jnp.int32),
            pltpu.VMEM((FILL, D), jnp.float32),
            pltpu.VMEM((1, D), jnp.float32),
            pltpu.VMEM_SHARED((NS, 16), jnp.int32),
            pltpu.SemaphoreType.DMA,
            pltpu.SemaphoreType.DMA,
        ],
    )


def _digit_lanes(vec):
    dm = jnp.where((vec >= 48) & (vec <= 57), 1, 0).astype(jnp.int32)
    return [dm[r] for r in range(16)]


def _fused_body(ids_hbm, table_hbm, out_hbm, ids_v, zidx_v, pub_v, puball_v,
                buf_v, fix_v, shared, gsem, ssem):
    c = lax.axis_index("c")
    s = lax.axis_index("s")
    wid = c * NS + s
    base = wid * B_PER_W
    row = wid // W_PER_ROW
    j0 = (wid % W_PER_ROW) * B_PER_W
    pltpu.sync_copy(ids_hbm.at[pl.ds(row, 1), pl.ds(j0, B_PER_W)], ids_v)
    zidx_v[pl.ds(0, 16)] = jnp.zeros((16,), jnp.int32)
    pltpu.async_copy(
        table_hbm.at[zidx_v.at[pl.ds(0, FILL)]], buf_v, gsem
    ).wait()
    fills = []
    for j in range(NFILL):
        fills.append(pltpu.async_copy(
            buf_v, out_hbm.at[pl.ds(base + j * FILL, FILL)], ssem
        ))

    def lastnd_group(g, last_nd):
        d = _digit_lanes(ids_v[0, pl.ds(g * 16, 16)])
        for r in range(16):
            last_nd = jnp.where(d[r] == 0, g * 16 + r, last_nd)
        return last_nd

    tail_run = (B_PER_W - 1) - lax.fori_loop(
        0, NGRP, lastnd_group, jnp.int32(-1)
    )
    lanes16 = lax.iota(jnp.int32, 16)
    pub_v[pl.ds(0, 16)] = jnp.where(lanes16 == 0, tail_run, 0)
    pltpu.sync_copy(pub_v, shared.at[s])
    plsc.subcore_barrier()
    pltpu.sync_copy(shared, puball_v)
    tails = [puball_v[j, pl.ds(0, 16)][0] for j in range(NS)]
    row_start = (s // W_PER_ROW) * W_PER_ROW
    carry = jnp.int32(0)
    alive = jnp.int32(1)
    for k in range(1, W_PER_ROW):
        pk = s - k
        spk = jnp.int32(0)
        for j in range(NS):
            spk = spk + jnp.where(pk == j, tails[j], 0)
        take = alive * jnp.where(pk >= row_start, 1, 0)
        carry = carry + take * spk
        alive = take * jnp.where(spk == B_PER_W, 1, 0)

    for f in fills:
        f.wait()

    def group(g, rl):
        vec = ids_v[0, pl.ds(g * 16, 16)]
        d = _digit_lanes(vec)
        any_d = d[0]
        for r in range(1, 16):
            any_d = any_d | d[r]
        pos = []
        for r in range(16):
            rl = d[r] * (rl + 1)
            pos.append(rl)

        @pl.when(any_d > 0)
        def _():
            for r in range(16):
                @pl.when(d[r] > 0)
                def _(r=r, p=pos[r]):
                    p = jnp.minimum(p, TABLE - 1)
                    pltpu.sync_copy(table_hbm.at[pl.ds(p, 1)], fix_v)
                    pltpu.sync_copy(
                        fix_v, out_hbm.at[pl.ds(base + g * 16 + r, 1)]
                    )
        return rl

    lax.fori_loop(0, NGRP, group, carry)


def kernel(input_ids, embedding):
    out = _build_fused()(input_ids, embedding)
    return out.reshape(B, L, D)


# speedup vs baseline: 1.1037x; 1.1037x over previous
import functools

import jax
import jax.numpy as jnp
from jax import lax
from jax.experimental import pallas as pl
from jax.experimental.pallas import tpu as pltpu
from jax.experimental.pallas import tpu_sc as plsc

B = 4
L = 4096
D = 1024
TABLE = 4096

NC = 2
NS = 16
NW = NC * NS
TOTAL = B * L
B_PER_W = TOTAL // NW
W_PER_ROW = L // B_PER_W
FILL = 4
NFILL = B_PER_W // FILL
NGRP = B_PER_W // 16


@functools.cache
def _build_fused():
    return pl.kernel(
        _fused_body,
        out_type=jax.ShapeDtypeStruct((TOTAL, D), jnp.float32),
        mesh=plsc.VectorSubcoreMesh(core_axis_name="c", subcore_axis_name="s"),
        scratch_types=[
            pltpu.VMEM((1, B_PER_W), jnp.int32),
            pltpu.VMEM((16,), jnp.int32),
            pltpu.VMEM((16,), jnp.int32),
            pltpu.VMEM((NS, 16), jnp.int32),
            pltpu.VMEM((FILL, D), jnp.float32),
            pltpu.VMEM((1, D), jnp.float32),
            pltpu.VMEM_SHARED((NS, 16), jnp.int32),
            pltpu.SemaphoreType.DMA,
            pltpu.SemaphoreType.DMA,
        ],
    )


def _digit_lanes(vec):
    dm = jnp.where((vec >= 48) & (vec <= 57), 1, 0).astype(jnp.int32)
    return [dm[r] for r in range(16)]


def _fused_body(ids_hbm, table_hbm, out_hbm, ids_v, zidx_v, pub_v, puball_v,
                buf_v, fix_v, shared, gsem, ssem):
    c = lax.axis_index("c")
    s = lax.axis_index("s")
    wid = c * NS + s
    base = wid * B_PER_W
    row = wid // W_PER_ROW
    j0 = (wid % W_PER_ROW) * B_PER_W
    ids_cp = pltpu.async_copy(
        ids_hbm.at[pl.ds(row, 1), pl.ds(j0, B_PER_W)], ids_v, gsem
    )
    zidx_v[pl.ds(0, 16)] = jnp.zeros((16,), jnp.int32)
    tmpl_cp = pltpu.async_copy(
        table_hbm.at[zidx_v.at[pl.ds(0, FILL)]], buf_v, gsem
    )
    ids_cp.wait()
    tmpl_cp.wait()
    fills = []
    for j in range(NFILL):
        fills.append(pltpu.async_copy(
            buf_v, out_hbm.at[pl.ds(base + j * FILL, FILL)], ssem
        ))

    def lastnd_group(g, last_nd):
        d = _digit_lanes(ids_v[0, pl.ds(g * 16, 16)])
        for r in range(16):
            last_nd = jnp.where(d[r] == 0, g * 16 + r, last_nd)
        return last_nd

    tail_run = (B_PER_W - 1) - lax.fori_loop(
        0, NGRP, lastnd_group, jnp.int32(-1)
    )
    lanes16 = lax.iota(jnp.int32, 16)
    pub_v[pl.ds(0, 16)] = jnp.where(lanes16 == 0, tail_run, 0)
    pltpu.sync_copy(pub_v, shared.at[s])
    plsc.subcore_barrier()
    pltpu.sync_copy(shared, puball_v)
    tails = [puball_v[j, pl.ds(0, 16)][0] for j in range(NS)]
    row_start = (s // W_PER_ROW) * W_PER_ROW
    carry = jnp.int32(0)
    alive = jnp.int32(1)
    for k in range(1, W_PER_ROW):
        pk = s - k
        spk = jnp.int32(0)
        for j in range(NS):
            spk = spk + jnp.where(pk == j, tails[j], 0)
        take = alive * jnp.where(pk >= row_start, 1, 0)
        carry = carry + take * spk
        alive = take * jnp.where(spk == B_PER_W, 1, 0)

    for f in fills:
        f.wait()

    def group(g, rl):
        vec = ids_v[0, pl.ds(g * 16, 16)]
        d = _digit_lanes(vec)
        any_d = d[0]
        for r in range(1, 16):
            any_d = any_d | d[r]
        pos = []
        for r in range(16):
            rl = d[r] * (rl + 1)
            pos.append(rl)

        @pl.when(any_d > 0)
        def _():
            for r in range(16):
                @pl.when(d[r] > 0)
                def _(r=r, p=pos[r]):
                    p = jnp.minimum(p, TABLE - 1)
                    pltpu.sync_copy(table_hbm.at[pl.ds(p, 1)], fix_v)
                    pltpu.sync_copy(
                        fix_v, out_hbm.at[pl.ds(base + g * 16 + r, 1)]
                    )
        return rl

    lax.fori_loop(0, NGRP, group, carry)


def kernel(input_ids, embedding):
    out = _build_fused()(input_ids, embedding)
    return out.reshape(B, L, D)


# speedup vs baseline: 1.1043x; 1.0006x over previous
import functools

import jax
import jax.numpy as jnp
from jax import lax
from jax.experimental import pallas as pl
from jax.experimental.pallas import tpu as pltpu
from jax.experimental.pallas import tpu_sc as plsc

B = 4
L = 4096
D = 1024
TABLE = 4096

NC = 2
NS = 16
NW = NC * NS
TOTAL = B * L
B_PER_W = TOTAL // NW
W_PER_ROW = L // B_PER_W
FILL = 4
NFILL = B_PER_W // FILL
NGRP = B_PER_W // 16


@functools.cache
def _build_fused():
    return pl.kernel(
        _fused_body,
        out_type=jax.ShapeDtypeStruct((TOTAL, D), jnp.float32),
        mesh=plsc.VectorSubcoreMesh(core_axis_name="c", subcore_axis_name="s"),
        scratch_types=[
            pltpu.VMEM((1, B_PER_W), jnp.int32),
            pltpu.VMEM((16,), jnp.int32),
            pltpu.VMEM((16,), jnp.int32),
            pltpu.VMEM((NS, 16), jnp.int32),
            pltpu.VMEM((FILL, D), jnp.float32),
            pltpu.VMEM((1, D), jnp.float32),
            pltpu.VMEM_SHARED((NS, 16), jnp.int32),
            pltpu.SemaphoreType.DMA,
            pltpu.SemaphoreType.DMA,
        ],
    )


def _digit_lanes(vec):
    dm = jnp.where((vec >= 48) & (vec <= 57), 1, 0).astype(jnp.int32)
    return [dm[r] for r in range(16)]


def _fused_body(ids_hbm, table_hbm, out_hbm, ids_v, zidx_v, pub_v, puball_v,
                buf_v, fix_v, shared, gsem, ssem):
    c = lax.axis_index("c")
    s = lax.axis_index("s")
    wid = c * NS + s
    base = wid * B_PER_W
    row = wid // W_PER_ROW
    j0 = (wid % W_PER_ROW) * B_PER_W
    ids_cp = pltpu.async_copy(
        ids_hbm.at[pl.ds(row, 1), pl.ds(j0, B_PER_W)], ids_v, gsem
    )
    zidx_v[pl.ds(0, 16)] = jnp.zeros((16,), jnp.int32)
    tmpl_cp = pltpu.async_copy(
        table_hbm.at[zidx_v.at[pl.ds(0, FILL)]], buf_v, gsem
    )
    tmpl_cp.wait()
    fills = []
    for j in range(NFILL):
        fills.append(pltpu.async_copy(
            buf_v, out_hbm.at[pl.ds(base + j * FILL, FILL)], ssem
        ))
    ids_cp.wait()

    def lastnd_group(g, last_nd):
        d = _digit_lanes(ids_v[0, pl.ds(g * 16, 16)])
        for r in range(16):
            last_nd = jnp.where(d[r] == 0, g * 16 + r, last_nd)
        return last_nd

    tail_run = (B_PER_W - 1) - lax.fori_loop(
        0, NGRP, lastnd_group, jnp.int32(-1)
    )
    lanes16 = lax.iota(jnp.int32, 16)
    pub_v[pl.ds(0, 16)] = jnp.where(lanes16 == 0, tail_run, 0)
    pltpu.sync_copy(pub_v, shared.at[s])
    plsc.subcore_barrier()
    pltpu.sync_copy(shared, puball_v)
    tails = [puball_v[j, pl.ds(0, 16)][0] for j in range(NS)]
    row_start = (s // W_PER_ROW) * W_PER_ROW
    carry = jnp.int32(0)
    alive = jnp.int32(1)
    for k in range(1, W_PER_ROW):
        pk = s - k
        spk = jnp.int32(0)
        for j in range(NS):
            spk = spk + jnp.where(pk == j, tails[j], 0)
        take = alive * jnp.where(pk >= row_start, 1, 0)
        carry = carry + take * spk
        alive = take * jnp.where(spk == B_PER_W, 1, 0)

    for f in fills:
        f.wait()

    def group(g, rl):
        vec = ids_v[0, pl.ds(g * 16, 16)]
        d = _digit_lanes(vec)
        any_d = d[0]
        for r in range(1, 16):
            any_d = any_d | d[r]
        pos = []
        for r in range(16):
            rl = d[r] * (rl + 1)
            pos.append(rl)

        @pl.when(any_d > 0)
        def _():
            for r in range(16):
                @pl.when(d[r] > 0)
                def _(r=r, p=pos[r]):
                    p = jnp.minimum(p, TABLE - 1)
                    pltpu.sync_copy(table_hbm.at[pl.ds(p, 1)], fix_v)
                    pltpu.sync_copy(
                        fix_v, out_hbm.at[pl.ds(base + g * 16 + r, 1)]
                    )
        return rl

    lax.fori_loop(0, NGRP, group, carry)


def kernel(input_ids, embedding):
    out = _build_fused()(input_ids, embedding)
    return out.reshape(B, L, D)
